# baseline (speedup 1.0000x reference)
import functools

import jax
import jax.numpy as jnp
from jax import lax
from jax.experimental import pallas as pl
from jax.experimental.pallas import tpu as pltpu
from jax.experimental.pallas import tpu_sc as plsc

_L = 16
_ISTREAM = 128


@functools.lru_cache(maxsize=None)
def _make_gather(n_table_rows, dim, n_cols, n_samples):
    info = plsc.get_sparse_core_info()
    nw = info.num_cores * info.num_subcores
    spw = n_samples // nw
    n_streams = spw // _ISTREAM
    assert spw % _ISTREAM == 0 and n_cols % 2 == 0 and dim % _L == 0
    mesh = plsc.VectorSubcoreMesh(core_axis_name="c", subcore_axis_name="s")

    @functools.partial(
        pl.kernel,
        mesh=mesh,
        compiler_params=pltpu.CompilerParams(
            use_tc_tiling_on_sc=False, needs_layout_passes=False),
        out_type=jax.ShapeDtypeStruct((n_cols, dim, n_samples), jnp.float32),
        scratch_types=[
            pltpu.VMEM((n_cols, spw), jnp.int32),
            pltpu.VMEM((spw, dim), jnp.float32),
            pltpu.VMEM((spw, dim), jnp.float32),
            pltpu.VMEM((dim * spw,), jnp.float32),
            pltpu.VMEM((dim * spw,), jnp.float32),
            pltpu.SemaphoreType.DMA,
            pltpu.SemaphoreType.DMA,
            pltpu.SemaphoreType.DMA,
            pltpu.SemaphoreType.DMA,
        ],
    )
    def gather_kernel(table_hbm, idxT_hbm, outT_hbm, idx_v,
                      ga, gb, ta, tb, gsa, gsb, wsa, wsb):
        wid = lax.axis_index("s") * info.num_cores + lax.axis_index("c")
        s0 = wid * spw
        pltpu.sync_copy(idxT_hbm.at[:, pl.ds(s0, spw)], idx_v)

        def fire_gathers(gbuf, gsem, c):
            for k in range(n_streams):
                pltpu.async_copy(
                    table_hbm.at[idx_v.at[c, pl.ds(k * _ISTREAM, _ISTREAM)]],
                    gbuf.at[pl.ds(k * _ISTREAM, _ISTREAM)],
                    gsem,
                )

        def wait_gathers(gbuf, gsem):
            pltpu.make_async_copy(
                table_hbm.at[pl.ds(0, spw)], gbuf, gsem).wait()

        def fire_write(tbuf, wsem, c):
            for d in range(dim):
                pltpu.async_copy(
                    tbuf.at[pl.ds(d * spw, spw)],
                    outT_hbm.at[c, d, pl.ds(s0, spw)], wsem)

        def wait_write(tbuf, wsem):
            pltpu.make_async_copy(
                tbuf, outT_hbm.at[0, 0, pl.ds(0, dim * spw)], wsem).wait()

        lane = lax.iota(jnp.int32, _L)
        d_idx = [(lane + h * _L) * spw for h in range(dim // _L)]

        def transpose(gbuf, tbuf):
            def blk(v, carry):
                for h in range(dim // _L):
                    vals = gbuf[v, pl.ds(h * _L, _L)]
                    plsc.store_scatter(tbuf, [d_idx[h] + v], vals)
                return carry
            lax.fori_loop(0, spw, blk, 0)

        fire_gathers(ga, gsa, 0)

        def body(i, carry):
            c0 = i * 2
            c1 = c0 + 1
            wait_gathers(ga, gsa)
            fire_gathers(gb, gsb, c1)

            @pl.when(i > 0)
            def _():
                wait_write(ta, wsa)
            transpose(ga, ta)
            fire_write(ta, wsa, c0)

            wait_gathers(gb, gsb)

            @pl.when(c1 + 1 < n_cols)
            def _():
                fire_gathers(ga, gsa, c1 + 1)

            @pl.when(i > 0)
            def _():
                wait_write(tb, wsb)
            transpose(gb, tb)
            fire_write(tb, wsb, c1)
            return carry

        lax.fori_loop(0, n_cols // 2, body, 0)
        wait_write(ta, wsa)
        wait_write(tb, wsb)

    return gather_kernel


def kernel(label, table):
    n_samples, n_cols = label.shape
    out = _make_gather(table.shape[0], table.shape[1],
                       n_cols, n_samples)(table, label.T)
    return out.transpose(2, 0, 1)
